# baseline (speedup 1.0000x reference)
import numpy as np
import jax
import jax.numpy as jnp
from jax import lax
from jax.experimental import pallas as pl
from jax.experimental.pallas import tpu as pltpu
from jax.experimental.pallas import tpu_sc as plsc

_WIN = 24
_R = 2 * _WIN - 1
_RP = _R + 1
_H = 32
_N = _WIN * _WIN
_CH = 48


def _build_coords_const() -> np.ndarray:
    d = np.arange(-(_WIN - 1), _WIN, dtype=np.float32)
    hh, ww = np.meshgrid(d, d, indexing="ij")
    tbl = np.stack([hh, ww], axis=-1).reshape(-1, 2)
    tbl = tbl / np.float32(_WIN - 1)
    tbl = tbl * 8.0
    tbl = np.sign(tbl) * np.log2(np.abs(tbl) + 1.0) / np.log2(8.0)
    tbl = tbl.astype(np.float32)

    flat = np.zeros((_R * _RP, 2), dtype=np.float32)
    a = np.arange(_R)
    b = np.arange(_R)
    cols = (a[:, None] * _RP + b[None, :]).reshape(-1)
    src = (2208 - (a[:, None] * _R + b[None, :])).reshape(-1)
    flat[cols] = tbl[src]
    return np.ascontiguousarray(flat.T)


_COORDS_CONST = _build_coords_const()

_BASES_CONST = np.stack(
    [v
     for k in range(_N // 16)
     for v in ((np.arange(16 * k, 16 * k + 16) // _WIN).astype(np.int32),
               (np.arange(16 * k, 16 * k + 16) % _WIN).astype(np.int32))]
)


def _tc_mlp_body(c_ref, w0_ref, b0_ref, w1_ref, d_ref):
    c = c_ref[...]
    w0t = w0_ref[...].T
    b0 = b0_ref[...].reshape(1, -1).T
    hid = (w0t[:, 0:1] * c[0:1, :] + w0t[:, 1:2] * c[1:2, :]
           + b0)
    hid = jnp.maximum(hid, 0.0)
    t = jnp.dot(w1_ref[...].T, hid, preferred_element_type=jnp.float32)
    t = 16.0 / (1.0 + jnp.exp(-t))
    d_ref[...] = t.reshape(_H, _R, _RP)


def _sc_expand_body(base_hbm, tbl_hbm, out_hbm, base_v, tbl_v, buf_v, sem):
    c = lax.axis_index("c")
    s = lax.axis_index("s")
    h = s * 2 + c

    pltpu.sync_copy(tbl_hbm.at[h], tbl_v)
    pltpu.sync_copy(base_hbm, base_v)

    def _build_row(mm, m0):
        m = m0 + mm
        ih = m // _WIN
        iw = m % _WIN
        ro = 23 - ih
        co = 23 - iw
        for k in range(_N // 16):
            jh = base_v[2 * k, pl.ds(0, 16)] + ro
            jw = base_v[2 * k + 1, pl.ds(0, 16)] + co
            buf_v[mm, pl.ds(16 * k, 16)] = plsc.load_gather(tbl_v, [jh, jw])
        return m0

    for chunk in range(_N // _CH):
        m0 = chunk * _CH
        lax.fori_loop(0, _CH, _build_row, m0)
        pltpu.make_async_copy(
            buf_v, out_hbm.at[h, pl.ds(m0, _CH)], sem).start()
        pltpu.make_async_copy(
            buf_v, out_hbm.at[h, pl.ds(m0, _CH)], sem).wait()


_SC_EXPAND_CACHE = []


def _sc_expand():
    if not _SC_EXPAND_CACHE:
        _SC_EXPAND_CACHE.append(pl.kernel(
            _sc_expand_body,
            out_type=jax.ShapeDtypeStruct((_H, _N, _N), jnp.float32),
            mesh=plsc.VectorSubcoreMesh(
                core_axis_name="c", subcore_axis_name="s"),
            scratch_types=[
                pltpu.VMEM((2 * _N // 16, 16), jnp.int32),
                pltpu.VMEM((_R, _RP), jnp.float32),
                pltpu.VMEM((_CH, _N), jnp.float32),
                pltpu.SemaphoreType.DMA,
            ],
            compiler_params=pltpu.CompilerParams(
                use_tc_tiling_on_sc=True, needs_layout_passes=False),
        ))
    return _SC_EXPAND_CACHE[0]


def kernel(W0, b0, W1):
    coords = jnp.asarray(_COORDS_CONST)
    table = pl.pallas_call(
        _tc_mlp_body,
        out_shape=jax.ShapeDtypeStruct((_H, _R, _RP), jnp.float32),
    )(coords, W0, b0, W1)
    return _sc_expand()(jnp.asarray(_BASES_CONST), table)


# speedup vs baseline: 3.0202x; 3.0202x over previous
import numpy as np
import jax
import jax.numpy as jnp
from jax import lax
from jax.experimental import pallas as pl
from jax.experimental.pallas import tpu as pltpu
from jax.experimental.pallas import tpu_sc as plsc

_WIN = 24
_R = 2 * _WIN - 1
_RP = _R + 1
_H = 32
_N = _WIN * _WIN


def _build_coords_const() -> np.ndarray:
    d = np.arange(-(_WIN - 1), _WIN, dtype=np.float32)
    hh, ww = np.meshgrid(d, d, indexing="ij")
    tbl = np.stack([hh, ww], axis=-1).reshape(-1, 2)
    tbl = tbl / np.float32(_WIN - 1)
    tbl = tbl * 8.0
    tbl = np.sign(tbl) * np.log2(np.abs(tbl) + 1.0) / np.log2(8.0)
    tbl = tbl.astype(np.float32)

    flat = np.zeros((_R * _RP, 2), dtype=np.float32)
    a = np.arange(_R)
    b = np.arange(_R)
    cols = (a[:, None] * _RP + b[None, :]).reshape(-1)
    src = (2208 - (a[:, None] * _R + b[None, :])).reshape(-1)
    flat[cols] = tbl[src]
    return np.ascontiguousarray(flat.T)


_COORDS_CONST = _build_coords_const()


def _tc_mlp_body(c_ref, w0_ref, b0_ref, w1_ref, d_ref):
    c = c_ref[...]
    w0t = w0_ref[...].T
    b0 = b0_ref[...].reshape(1, -1).T
    hid = (w0t[:, 0:1] * c[0:1, :] + w0t[:, 1:2] * c[1:2, :]
           + b0)
    hid = jnp.maximum(hid, 0.0)
    t = jnp.dot(w1_ref[...].T, hid, preferred_element_type=jnp.float32)
    t = 16.0 / (1.0 + jnp.exp(-t))
    d_ref[...] = t.reshape(_H, _R, _RP)


def _sc_expand_body(tbl_hbm, out_hbm, tbl_v, d_v, sem):
    c = lax.axis_index("c")
    s = lax.axis_index("s")
    h = s * 2 + c

    pltpu.sync_copy(tbl_hbm.at[h], tbl_v)

    def _build_row(a, carry):
        for iw in range(_WIN):
            o = 23 - iw
            d_v[iw, a, pl.ds(0, 16)] = tbl_v[a, pl.ds(o, 16)]
            d_v[iw, a, pl.ds(8, 16)] = tbl_v[a, pl.ds(o + 8, 16)]
        return carry

    lax.fori_loop(0, _R, _build_row, 0)

    for ih in range(_WIN):
        pltpu.make_async_copy(
            d_v.at[:, pl.ds(23 - ih, _WIN), :], out_hbm.at[h, ih], sem
        ).start()
    for ih in range(_WIN):
        pltpu.make_async_copy(
            d_v.at[:, pl.ds(23 - ih, _WIN), :], out_hbm.at[h, ih], sem
        ).wait()


_SC_EXPAND_CACHE = []


def _sc_expand():
    if not _SC_EXPAND_CACHE:
        _SC_EXPAND_CACHE.append(pl.kernel(
            _sc_expand_body,
            out_type=jax.ShapeDtypeStruct(
                (_H, _WIN, _WIN, _WIN, _WIN), jnp.float32),
            mesh=plsc.VectorSubcoreMesh(
                core_axis_name="c", subcore_axis_name="s"),
            scratch_types=[
                pltpu.VMEM((_R, _RP), jnp.float32),
                pltpu.VMEM((_WIN, _R, _WIN), jnp.float32),
                pltpu.SemaphoreType.DMA,
            ],
            compiler_params=pltpu.CompilerParams(use_tc_tiling_on_sc=False),
        ))
    return _SC_EXPAND_CACHE[0]


def kernel(W0, b0, W1):
    coords = jnp.asarray(_COORDS_CONST)
    table = pl.pallas_call(
        _tc_mlp_body,
        out_shape=jax.ShapeDtypeStruct((_H, _R, _RP), jnp.float32),
    )(coords, W0, b0, W1)
    out5 = _sc_expand()(table)
    return out5.reshape(_H, _N, _N)
